# Initial kernel scaffold; baseline (speedup 1.0000x reference)
import jax
import jax.numpy as jnp
from jax.experimental import pallas as pl


def kernel(atom_fea, edge_ij, nbr_atoms, bonds_r, W1, b1, W2, b2, Wr, br, W3, b3):
    raise NotImplementedError("write your pallas kernel here")


# speedup vs baseline: 3.0167x; 3.0167x over previous
import functools

import jax
import jax.numpy as jnp
from jax import lax
from jax.experimental import pallas as pl
from jax.experimental.pallas import tpu as pltpu
from jax.experimental.pallas import tpu_sc as plsc


def _proj_body(a_ref, wa_ref, wb_ref, pa_ref, pb_ref):
    a = a_ref[...]
    pa_ref[...] = jnp.dot(a, wa_ref[...], preferred_element_type=jnp.float32)
    pb_ref[...] = jnp.dot(a, wb_ref[...], preferred_element_type=jnp.float32)


def _node_proj(atom_fea, wa, wb):
    n = atom_fea.shape[0]
    d = wa.shape[1]
    return pl.pallas_call(
        _proj_body,
        out_shape=[jax.ShapeDtypeStruct((n, d), jnp.float32)] * 2,
    )(atom_fea, wa, wb)


def _gather_sum_sc(pa, pb, idx0, idx1):
    e_total = idx0.shape[0]
    d = pa.shape[1]
    info = plsc.get_sparse_core_info()
    nc, ns = info.num_cores, info.num_subcores
    nw = nc * ns
    epw = e_total // nw
    chunk = 1000
    n_chunks = epw // chunk
    assert epw * nw == e_total and chunk * n_chunks == epw
    mesh = plsc.VectorSubcoreMesh(core_axis_name="c", subcore_axis_name="s")

    @functools.partial(
        pl.kernel,
        mesh=mesh,
        out_type=jax.ShapeDtypeStruct((e_total, d), jnp.float32),
        scratch_types=[
            pltpu.VMEM((chunk,), jnp.int32),
            pltpu.VMEM((chunk,), jnp.int32),
            pltpu.VMEM((chunk, d), jnp.float32),
            pltpu.VMEM((chunk, d), jnp.float32),
            pltpu.SemaphoreType.DMA,
            pltpu.SemaphoreType.DMA,
        ],
        compiler_params=pltpu.CompilerParams(use_tc_tiling_on_sc=False),
    )
    def k(pa_hbm, pb_hbm, i0_hbm, i1_hbm, o_hbm, i0_v, i1_v, ga_v, gb_v, sa, sb):
        wid = lax.axis_index("s") * nc + lax.axis_index("c")
        base = wid * epw

        @pl.loop(0, n_chunks)
        def _(t):
            off = base + t * chunk
            pltpu.sync_copy(i0_hbm.at[pl.ds(off, chunk)], i0_v)
            pltpu.sync_copy(i1_hbm.at[pl.ds(off, chunk)], i1_v)
            ca = pltpu.async_copy(pa_hbm.at[i0_v], ga_v, sa)
            cb = pltpu.async_copy(pb_hbm.at[i1_v], gb_v, sb)
            ca.wait()
            cb.wait()

            @pl.loop(0, chunk)
            def _(r):
                for c in range(0, d, 16):
                    slc = (pl.ds(r, 1), pl.ds(c, 16))
                    ga_v.at[*slc][...] = ga_v.at[*slc][...] + gb_v.at[*slc][...]

            pltpu.sync_copy(ga_v, o_hbm.at[pl.ds(off, chunk)])

    return k(pa, pb, idx0, idx1)


def _edge_body(s_ref, e_ref, r_ref, w1c_ref, w2c_ref, w3_ref, wr_ref, bias_ref,
               o_ref):
    s = s_ref[...]
    e = e_ref[...]
    r = r_ref[...]
    b1 = bias_ref[0:1, :]
    b2 = bias_ref[1:2, :]
    b3 = bias_ref[2:3, :]
    brr = bias_ref[3:4, :]
    z1 = s[:, :16] + jnp.dot(e, w1c_ref[...], preferred_element_type=jnp.float32) + b1
    z2 = s[:, 16:] + jnp.dot(e, w2c_ref[...], preferred_element_type=jnp.float32) + b2
    h = (z1 * jax.nn.sigmoid(z1)) * jax.nn.sigmoid(z2)
    t = jnp.dot(h, w3_ref[...], preferred_element_type=jnp.float32) + b3
    g = jnp.dot(r, wr_ref[...], preferred_element_type=jnp.float32) + brr
    o_ref[...] = (t * jax.nn.sigmoid(t)) * g


def _edge_mlp(s, edge_ij, bonds_r, w1c_t, w2c_t, w3_t, wr_t, bias):
    e_total = s.shape[0]
    be = 8000
    assert e_total % be == 0
    small = lambda shp: pl.BlockSpec(shp, lambda i: (0, 0))
    return pl.pallas_call(
        _edge_body,
        grid=(e_total // be,),
        in_specs=[
            pl.BlockSpec((be, 32), lambda i: (i, 0)),
            pl.BlockSpec((be, 16), lambda i: (i, 0)),
            pl.BlockSpec((be, 16), lambda i: (i, 0)),
            small((16, 16)),
            small((16, 16)),
            small((16, 16)),
            small((16, 16)),
            small((4, 16)),
        ],
        out_specs=pl.BlockSpec((be, 16), lambda i: (i, 0)),
        out_shape=jax.ShapeDtypeStruct((e_total, 16), jnp.float32),
    )(s, edge_ij, bonds_r, w1c_t, w2c_t, w3_t, wr_t, bias)


def kernel(atom_fea, edge_ij, nbr_atoms, bonds_r, W1, b1, W2, b2, Wr, br, W3, b3):
    f = atom_fea.shape[1]
    wa = jnp.concatenate([W1[:, :f].T, W2[:, :f].T], axis=1)
    wb = jnp.concatenate([W1[:, f:2 * f].T, W2[:, f:2 * f].T], axis=1)
    w1c_t = W1[:, 2 * f:].T
    w2c_t = W2[:, 2 * f:].T
    w3_t = W3.T
    wr_t = Wr.T
    bias = jnp.stack([b1, b2, b3, br], axis=0)
    idx0 = nbr_atoms[:, 0]
    idx1 = nbr_atoms[:, 1]

    pa, pb = _node_proj(atom_fea, wa, wb)
    s = _gather_sum_sc(pa, pb, idx0, idx1)
    return _edge_mlp(s, edge_ij, bonds_r, w1c_t, w2c_t, w3_t, wr_t, bias)


# speedup vs baseline: 3.0615x; 1.0148x over previous
import functools

import jax
import jax.numpy as jnp
from jax import lax
from jax.experimental import pallas as pl
from jax.experimental.pallas import tpu as pltpu
from jax.experimental.pallas import tpu_sc as plsc


def _proj_body(a_ref, wa_ref, wb_ref, pa_ref, pb_ref):
    a = a_ref[...]
    pa_ref[...] = jnp.dot(a, wa_ref[...], preferred_element_type=jnp.float32)
    pb_ref[...] = jnp.dot(a, wb_ref[...], preferred_element_type=jnp.float32)


def _node_proj(atom_fea, wa, wb):
    n = atom_fea.shape[0]
    d = wa.shape[1]
    return pl.pallas_call(
        _proj_body,
        out_shape=[jax.ShapeDtypeStruct((n, d), jnp.float32)] * 2,
    )(atom_fea, wa, wb)


def _gather_sum_sc(pa, pb, idx0, idx1):
    e_total = idx0.shape[0]
    d = pa.shape[1]
    info = plsc.get_sparse_core_info()
    nc, ns = info.num_cores, info.num_subcores
    nw = nc * ns
    epw = e_total // nw
    chunk = 1000
    n_chunks = epw // chunk
    assert epw * nw == e_total and chunk * n_chunks == epw
    mesh = plsc.VectorSubcoreMesh(core_axis_name="c", subcore_axis_name="s")

    @functools.partial(
        pl.kernel,
        mesh=mesh,
        out_type=jax.ShapeDtypeStruct((e_total, d), jnp.float32),
        scratch_types=[
            pltpu.VMEM((chunk,), jnp.int32),
            pltpu.VMEM((chunk,), jnp.int32),
            pltpu.VMEM((chunk, d), jnp.float32),
            pltpu.VMEM((chunk, d), jnp.float32),
            pltpu.SemaphoreType.DMA,
            pltpu.SemaphoreType.DMA,
        ],
        compiler_params=pltpu.CompilerParams(use_tc_tiling_on_sc=False),
    )
    def k(pa_hbm, pb_hbm, i0_hbm, i1_hbm, o_hbm, i0_v, i1_v, ga_v, gb_v, sa, sb):
        wid = lax.axis_index("s") * nc + lax.axis_index("c")
        base = wid * epw

        @pl.loop(0, n_chunks)
        def _(t):
            off = base + t * chunk
            pltpu.sync_copy(i0_hbm.at[pl.ds(off, chunk)], i0_v)
            pltpu.sync_copy(i1_hbm.at[pl.ds(off, chunk)], i1_v)
            ca = pltpu.async_copy(pa_hbm.at[i0_v], ga_v, sa)
            cb = pltpu.async_copy(pb_hbm.at[i1_v], gb_v, sb)
            ca.wait()
            cb.wait()

            @pl.loop(0, chunk)
            def _(r):
                for c in range(0, d, 16):
                    slc = (pl.ds(r, 1), pl.ds(c, 16))
                    ga_v.at[*slc][...] = ga_v.at[*slc][...] + gb_v.at[*slc][...]

            pltpu.sync_copy(ga_v, o_hbm.at[pl.ds(off, chunk)])

    return k(pa, pb, idx0, idx1)


def _edge_body(s_ref, e_ref, r_ref, w1c_ref, w2c_ref, w3_ref, wr_ref, bias_ref,
               o_ref):
    st = s_ref[...].T
    et = e_ref[...].T
    rt = r_ref[...].T
    b1 = bias_ref[:, 0:1]
    b2 = bias_ref[:, 1:2]
    b3 = bias_ref[:, 2:3]
    brr = bias_ref[:, 3:4]
    z1 = st[:16] + jnp.dot(w1c_ref[...], et, preferred_element_type=jnp.float32) + b1
    z2 = st[16:] + jnp.dot(w2c_ref[...], et, preferred_element_type=jnp.float32) + b2
    h = (z1 * jax.nn.sigmoid(z1)) * jax.nn.sigmoid(z2)
    t = jnp.dot(w3_ref[...], h, preferred_element_type=jnp.float32) + b3
    g = jnp.dot(wr_ref[...], rt, preferred_element_type=jnp.float32) + brr
    o_ref[...] = ((t * jax.nn.sigmoid(t)) * g).T


def _edge_mlp(s, edge_ij, bonds_r, w1c, w2c, w3, wr, bias):
    e_total = s.shape[0]
    be = 12800
    assert e_total % be == 0
    small = lambda shp: pl.BlockSpec(shp, lambda i: (0, 0))
    return pl.pallas_call(
        _edge_body,
        grid=(e_total // be,),
        in_specs=[
            pl.BlockSpec((be, 32), lambda i: (i, 0)),
            pl.BlockSpec((be, 16), lambda i: (i, 0)),
            pl.BlockSpec((be, 16), lambda i: (i, 0)),
            small((16, 16)),
            small((16, 16)),
            small((16, 16)),
            small((16, 16)),
            small((16, 4)),
        ],
        out_specs=pl.BlockSpec((be, 16), lambda i: (i, 0)),
        out_shape=jax.ShapeDtypeStruct((e_total, 16), jnp.float32),
    )(s, edge_ij, bonds_r, w1c, w2c, w3, wr, bias)


def kernel(atom_fea, edge_ij, nbr_atoms, bonds_r, W1, b1, W2, b2, Wr, br, W3, b3):
    f = atom_fea.shape[1]
    wa = jnp.concatenate([W1[:, :f].T, W2[:, :f].T], axis=1)
    wb = jnp.concatenate([W1[:, f:2 * f].T, W2[:, f:2 * f].T], axis=1)
    w1c = W1[:, 2 * f:]
    w2c = W2[:, 2 * f:]
    bias = jnp.stack([b1, b2, b3, br], axis=1)
    idx0 = nbr_atoms[:, 0]
    idx1 = nbr_atoms[:, 1]

    pa, pb = _node_proj(atom_fea, wa, wb)
    s = _gather_sum_sc(pa, pb, idx0, idx1)
    return _edge_mlp(s, edge_ij, bonds_r, w1c, w2c, W3, Wr, bias)


# speedup vs baseline: 6.0791x; 1.9857x over previous
import functools

import jax
import jax.numpy as jnp
from jax import lax
from jax.experimental import pallas as pl
from jax.experimental.pallas import tpu as pltpu
from jax.experimental.pallas import tpu_sc as plsc


def _proj_body(a_ref, wa_ref, wb_ref, pa_ref, pb_ref):
    a = a_ref[...]
    pa_ref[...] = jnp.dot(a, wa_ref[...], preferred_element_type=jnp.float32)
    pb_ref[...] = jnp.dot(a, wb_ref[...], preferred_element_type=jnp.float32)


def _node_proj(atom_fea, wa, wb):
    n = atom_fea.shape[0]
    d = wa.shape[1]
    return pl.pallas_call(
        _proj_body,
        out_shape=[jax.ShapeDtypeStruct((n, d), jnp.float32)] * 2,
    )(atom_fea, wa, wb)


def _gather_sum_sc(pa, pb, idx0, idx1):
    e_total = idx0.shape[0]
    d = pa.shape[1]
    info = plsc.get_sparse_core_info()
    nc, ns = info.num_cores, info.num_subcores
    nw = nc * ns
    epw = e_total // nw
    chunk = 1000
    n_chunks = epw // chunk
    assert epw * nw == e_total and chunk * n_chunks == epw
    mesh = plsc.VectorSubcoreMesh(core_axis_name="c", subcore_axis_name="s")

    @functools.partial(
        pl.kernel,
        mesh=mesh,
        out_type=jax.ShapeDtypeStruct((e_total, d), jnp.float32),
        scratch_types=[
            pltpu.VMEM((chunk,), jnp.int32),
            pltpu.VMEM((chunk,), jnp.int32),
            pltpu.VMEM((chunk, d), jnp.float32),
            pltpu.VMEM((chunk, d), jnp.float32),
            pltpu.SemaphoreType.DMA,
            pltpu.SemaphoreType.DMA,
        ],
        compiler_params=pltpu.CompilerParams(use_tc_tiling_on_sc=False),
    )
    def k(pa_hbm, pb_hbm, i0_hbm, i1_hbm, o_hbm, i0_v, i1_v, ga_v, gb_v, sa, sb):
        wid = lax.axis_index("s") * nc + lax.axis_index("c")
        base = wid * epw

        @pl.loop(0, n_chunks)
        def _(t):
            off = base + t * chunk
            pltpu.sync_copy(i0_hbm.at[pl.ds(off, chunk)], i0_v)
            pltpu.sync_copy(i1_hbm.at[pl.ds(off, chunk)], i1_v)
            ca = pltpu.async_copy(pa_hbm.at[i0_v], ga_v, sa)
            cb = pltpu.async_copy(pb_hbm.at[i1_v], gb_v, sb)
            ca.wait()
            cb.wait()

            @pl.loop(0, chunk)
            def _(r):
                for c in range(0, d, 16):
                    slc = (pl.ds(r, 1), pl.ds(c, 16))
                    ga_v.at[*slc][...] = ga_v.at[*slc][...] + gb_v.at[*slc][...]

            pltpu.sync_copy(ga_v, o_hbm.at[pl.ds(off, chunk)])

    return k(pa, pb, idx0, idx1)


def _edge_body(s_ref, et_ref, rt_ref, w12_ref, w3_ref, wr_ref, bias_ref,
               o_ref):
    s2t = s_ref[...].T
    b12 = bias_ref[0:32]
    b3 = bias_ref[32:48]
    br4 = bias_ref[48:64]
    bc = s_ref.shape[0]
    for g in range(4):
        stg = s2t[32 * g:32 * g + 32]
        etg = et_ref[:, g * bc:(g + 1) * bc]
        rtg = rt_ref[:, g * bc:(g + 1) * bc]
        z12 = stg + jnp.dot(w12_ref[...], etg,
                            preferred_element_type=jnp.float32) + b12
        z1 = z12[:16]
        z2 = z12[16:]
        h = (z1 * jax.nn.sigmoid(z1)) * jax.nn.sigmoid(z2)
        t = jnp.dot(w3_ref[...], h, preferred_element_type=jnp.float32) + b3
        gg = jnp.dot(wr_ref[...], rtg, preferred_element_type=jnp.float32) + br4
        o_ref[:, g * bc:(g + 1) * bc] = (t * jax.nn.sigmoid(t)) * gg


def _edge_mlp_t(s128, et, rt, w12, w3, wr, bias_col):
    e_total = et.shape[1]
    be = 12800
    bc = be // 4
    nb = e_total // be
    assert e_total % be == 0
    small = lambda shp: pl.BlockSpec(shp, lambda b: (0, 0))
    return pl.pallas_call(
        _edge_body,
        grid=(nb,),
        in_specs=[
            pl.BlockSpec((bc, 128), lambda b: (b, 0)),
            pl.BlockSpec((16, be), lambda b: (0, b)),
            pl.BlockSpec((16, be), lambda b: (0, b)),
            small((32, 16)),
            small((16, 16)),
            small((16, 16)),
            small((64, 1)),
        ],
        out_specs=pl.BlockSpec((16, be), lambda b: (0, b)),
        out_shape=jax.ShapeDtypeStruct((16, e_total), jnp.float32),
    )(s128, et, rt, w12, w3, wr, bias_col)


def kernel(atom_fea, edge_ij, nbr_atoms, bonds_r, W1, b1, W2, b2, Wr, br, W3, b3):
    f = atom_fea.shape[1]
    e_total = edge_ij.shape[0]
    eq = e_total // 4
    wa = jnp.concatenate([W1[:, :f].T, W2[:, :f].T], axis=1)
    wb = jnp.concatenate([W1[:, f:2 * f].T, W2[:, f:2 * f].T], axis=1)
    w12 = jnp.concatenate([W1[:, 2 * f:], W2[:, 2 * f:]], axis=0)
    bias_col = jnp.concatenate([b1, b2, b3, br])[:, None]
    idx0 = nbr_atoms[:, 0].reshape(-1, 4, 3200).transpose(0, 2, 1).reshape(-1)
    idx1 = nbr_atoms[:, 1].reshape(-1, 4, 3200).transpose(0, 2, 1).reshape(-1)

    pa, pb = _node_proj(atom_fea, wa, wb)
    s = _gather_sum_sc(pa, pb, idx0, idx1)
    s128 = s.reshape(-1, 128)
    ot = _edge_mlp_t(s128, edge_ij.T, bonds_r.T, w12, W3, Wr, bias_col)
    return ot.T


# speedup vs baseline: 8.5940x; 1.4137x over previous
import functools

import jax
import jax.numpy as jnp
from jax import lax
from jax.experimental import pallas as pl
from jax.experimental.pallas import tpu as pltpu
from jax.experimental.pallas import tpu_sc as plsc


def _proj_body(a_ref, wa_ref, wb_ref, pa_ref, pb_ref):
    a = a_ref[...]
    pa_ref[...] = jnp.dot(a, wa_ref[...], preferred_element_type=jnp.float32)
    pb_ref[...] = jnp.dot(a, wb_ref[...], preferred_element_type=jnp.float32)


def _node_proj(atom_fea, wa, wb):
    n = atom_fea.shape[0]
    d = wa.shape[1]
    return pl.pallas_call(
        _proj_body,
        out_shape=[jax.ShapeDtypeStruct((n, d), jnp.float32)] * 2,
    )(atom_fea, wa, wb)


def _gather_sum_sc(pa, pb, idx0, idx1):
    e_total = idx0.shape[0]
    d = pa.shape[1]
    info = plsc.get_sparse_core_info()
    nc, ns = info.num_cores, info.num_subcores
    nw = nc * ns
    unit = 3200
    n_units = e_total // unit
    chunk = 1600
    n_sub = unit // chunk
    max_u = (n_units + nw - 1) // nw
    assert n_units * unit == e_total
    mesh = plsc.VectorSubcoreMesh(core_axis_name="c", subcore_axis_name="s")

    @functools.partial(
        pl.kernel,
        mesh=mesh,
        out_type=jax.ShapeDtypeStruct((e_total // 4, 4 * d), jnp.float32),
        scratch_types=[
            pltpu.VMEM((unit,), jnp.int32),
            pltpu.VMEM((unit,), jnp.int32),
            pltpu.VMEM((chunk, d), jnp.float32),
            pltpu.VMEM((chunk, d), jnp.float32),
            pltpu.SemaphoreType.DMA,
            pltpu.SemaphoreType.DMA,
        ],
        compiler_params=pltpu.CompilerParams(use_tc_tiling_on_sc=False),
    )
    def k(pa_hbm, pb_hbm, i0_hbm, i1_hbm, o_hbm, i0_v, i1_v, ga_v, gb_v, sa, sb):
        wid = lax.axis_index("s") * nc + lax.axis_index("c")
        for u in range(max_u):
            uid = wid + nw * u

            @pl.when(uid < n_units)
            def _():
                b = uid // 4
                g = lax.rem(uid, 4)
                base_e = uid * unit

                pltpu.sync_copy(i0_hbm.at[pl.ds(base_e, unit)], i0_v)
                pltpu.sync_copy(i1_hbm.at[pl.ds(base_e, unit)], i1_v)
                for sub in range(n_sub):
                    ca = pltpu.async_copy(
                        pa_hbm.at[i0_v.at[pl.ds(sub * chunk, chunk)]], ga_v, sa)
                    cb = pltpu.async_copy(
                        pb_hbm.at[i1_v.at[pl.ds(sub * chunk, chunk)]], gb_v, sb)
                    ca.wait()
                    cb.wait()

                    @pl.loop(0, chunk)
                    def _(r):
                        for c in range(0, d, 16):
                            slc = (pl.ds(r, 1), pl.ds(c, 16))
                            ga_v.at[*slc][...] = (ga_v.at[*slc][...]
                                                  + gb_v.at[*slc][...])

                    pltpu.sync_copy(
                        ga_v,
                        o_hbm.at[pl.ds(b * unit + sub * chunk, chunk),
                                 pl.ds(g * d, d)])

    return k(pa, pb, idx0, idx1)


def _edge_body(s_ref, et_ref, rt_ref, w12_ref, w3_ref, wr_ref, bias_ref,
               o_ref):
    s2t = s_ref[...].T
    b12 = bias_ref[0:32]
    b3 = bias_ref[32:48]
    br4 = bias_ref[48:64]
    bc = s_ref.shape[0]
    for g in range(4):
        stg = s2t[32 * g:32 * g + 32]
        etg = et_ref[:, g * bc:(g + 1) * bc]
        rtg = rt_ref[:, g * bc:(g + 1) * bc]
        z12 = stg + jnp.dot(w12_ref[...], etg,
                            preferred_element_type=jnp.float32) + b12
        z1 = z12[:16]
        z2 = z12[16:]
        h = (z1 * jax.nn.sigmoid(z1)) * jax.nn.sigmoid(z2)
        t = jnp.dot(w3_ref[...], h, preferred_element_type=jnp.float32) + b3
        gg = jnp.dot(wr_ref[...], rtg, preferred_element_type=jnp.float32) + br4
        o_ref[:, g * bc:(g + 1) * bc] = (t * jax.nn.sigmoid(t)) * gg


def _edge_mlp_t(s128, et, rt, w12, w3, wr, bias_col):
    e_total = et.shape[1]
    be = 12800
    bc = be // 4
    nb = e_total // be
    assert e_total % be == 0
    small = lambda shp: pl.BlockSpec(shp, lambda b: (0, 0))
    return pl.pallas_call(
        _edge_body,
        grid=(nb,),
        in_specs=[
            pl.BlockSpec((bc, 128), lambda b: (b, 0)),
            pl.BlockSpec((16, be), lambda b: (0, b)),
            pl.BlockSpec((16, be), lambda b: (0, b)),
            small((32, 16)),
            small((16, 16)),
            small((16, 16)),
            small((64, 1)),
        ],
        out_specs=pl.BlockSpec((16, be), lambda b: (0, b)),
        out_shape=jax.ShapeDtypeStruct((16, e_total), jnp.float32),
    )(s128, et, rt, w12, w3, wr, bias_col)


def kernel(atom_fea, edge_ij, nbr_atoms, bonds_r, W1, b1, W2, b2, Wr, br, W3, b3):
    f = atom_fea.shape[1]
    e_total = edge_ij.shape[0]
    eq = e_total // 4
    wa = jnp.concatenate([W1[:, :f].T, W2[:, :f].T], axis=1)
    wb = jnp.concatenate([W1[:, f:2 * f].T, W2[:, f:2 * f].T], axis=1)
    w12 = jnp.concatenate([W1[:, 2 * f:], W2[:, 2 * f:]], axis=0)
    bias_col = jnp.concatenate([b1, b2, b3, br])[:, None]
    idx0 = nbr_atoms[:, 0]
    idx1 = nbr_atoms[:, 1]

    pa, pb = _node_proj(atom_fea, wa, wb)
    s128 = _gather_sum_sc(pa, pb, idx0, idx1)
    ot = _edge_mlp_t(s128, edge_ij.T, bonds_r.T, w12, W3, Wr, bias_col)
    return ot.T


# speedup vs baseline: 10.2480x; 1.1925x over previous
import functools

import jax
import jax.numpy as jnp
from jax import lax
from jax.experimental import pallas as pl
from jax.experimental.pallas import tpu as pltpu
from jax.experimental.pallas import tpu_sc as plsc


def _proj_body(a_ref, wa_ref, wb_ref, pa_ref, pb_ref):
    a = a_ref[...]
    pa_ref[...] = jnp.dot(a, wa_ref[...], preferred_element_type=jnp.float32)
    pb_ref[...] = jnp.dot(a, wb_ref[...], preferred_element_type=jnp.float32)


def _node_proj(atom_fea, wa, wb):
    n = atom_fea.shape[0]
    d = wa.shape[1]
    return pl.pallas_call(
        _proj_body,
        out_shape=[jax.ShapeDtypeStruct((n, d), jnp.float32)] * 2,
    )(atom_fea, wa, wb)


def _gather_sum_sc(pa, pb, nbr_t):
    e_total = nbr_t.shape[1]
    d = pa.shape[1]
    info = plsc.get_sparse_core_info()
    nc, ns = info.num_cores, info.num_subcores
    nw = nc * ns
    unit = 3200
    n_units = e_total // unit
    chunk = 800
    n_sub = unit // chunk
    max_u = (n_units + nw - 1) // nw
    assert n_units * unit == e_total
    mesh = plsc.VectorSubcoreMesh(core_axis_name="c", subcore_axis_name="s")

    @functools.partial(
        pl.kernel,
        mesh=mesh,
        out_type=jax.ShapeDtypeStruct((e_total // 4, 4 * d), jnp.float32),
        scratch_types=[
            pltpu.VMEM((unit,), jnp.int32),
            pltpu.VMEM((unit,), jnp.int32),
            [pltpu.VMEM((chunk, d), jnp.float32) for _ in range(2)],
            [pltpu.VMEM((chunk, d), jnp.float32) for _ in range(2)],
            [pltpu.SemaphoreType.DMA for _ in range(2)],
            [pltpu.SemaphoreType.DMA for _ in range(2)],
            [pltpu.SemaphoreType.DMA for _ in range(2)],
        ],
        compiler_params=pltpu.CompilerParams(use_tc_tiling_on_sc=False),
    )
    def k(pa_hbm, pb_hbm, nbr_hbm, o_hbm, i0_v, i1_v, ga, gb, sa, sb, sw):
        wid = lax.axis_index("s") * nc + lax.axis_index("c")

        for u in range(max_u):
            uid = wid + nw * u

            @pl.when(uid < n_units)
            def _():
                b = uid // 4
                g = lax.rem(uid, 4)
                base_e = uid * unit

                pltpu.sync_copy(nbr_hbm.at[0, pl.ds(base_e, unit)], i0_v)
                pltpu.sync_copy(nbr_hbm.at[1, pl.ds(base_e, unit)], i1_v)

                def start_gather(sub):
                    st = sub % 2
                    ca = pltpu.async_copy(
                        pa_hbm.at[i0_v.at[pl.ds(sub * chunk, chunk)]],
                        ga[st], sa[st])
                    cb = pltpu.async_copy(
                        pb_hbm.at[i1_v.at[pl.ds(sub * chunk, chunk)]],
                        gb[st], sb[st])
                    return ca, cb

                gops = start_gather(0)
                wops = [None, None]
                for sub in range(n_sub):
                    st = sub % 2
                    nxt = None
                    if sub + 1 < n_sub:
                        if wops[(sub + 1) % 2] is not None:
                            wops[(sub + 1) % 2].wait()
                            wops[(sub + 1) % 2] = None
                        nxt = start_gather(sub + 1)
                    gops[0].wait()
                    gops[1].wait()
                    gops = nxt

                    ga_v, gb_v = ga[st], gb[st]

                    @pl.loop(0, chunk)
                    def _(r):
                        for c in range(0, d, 16):
                            slc = (pl.ds(r, 1), pl.ds(c, 16))
                            ga_v.at[*slc][...] = (ga_v.at[*slc][...]
                                                  + gb_v.at[*slc][...])

                    wops[st] = pltpu.async_copy(
                        ga_v,
                        o_hbm.at[pl.ds(b * unit + sub * chunk, chunk),
                                 pl.ds(g * d, d)],
                        sw[st])
                for w in wops:
                    if w is not None:
                        w.wait()

    return k(pa, pb, nbr_t)


def _edge_body(s_ref, et_ref, rt_ref, w12_ref, w3_ref, wr_ref, bias_ref,
               o_ref):
    s2t = s_ref[...].T
    b12 = bias_ref[0:32]
    b3 = bias_ref[32:48]
    br4 = bias_ref[48:64]
    bc = s_ref.shape[0]
    for g in range(4):
        stg = s2t[32 * g:32 * g + 32]
        etg = et_ref[:, g * bc:(g + 1) * bc]
        rtg = rt_ref[:, g * bc:(g + 1) * bc]
        z12 = stg + jnp.dot(w12_ref[...], etg,
                            preferred_element_type=jnp.float32) + b12
        z1 = z12[:16]
        z2 = z12[16:]
        h = (z1 * jax.nn.sigmoid(z1)) * jax.nn.sigmoid(z2)
        t = jnp.dot(w3_ref[...], h, preferred_element_type=jnp.float32) + b3
        gg = jnp.dot(wr_ref[...], rtg, preferred_element_type=jnp.float32) + br4
        o_ref[:, g * bc:(g + 1) * bc] = (t * jax.nn.sigmoid(t)) * gg


def _edge_mlp_t(s128, et, rt, w12, w3, wr, bias_col):
    e_total = et.shape[1]
    be = 12800
    bc = be // 4
    nb = e_total // be
    assert e_total % be == 0
    small = lambda shp: pl.BlockSpec(shp, lambda b: (0, 0))
    return pl.pallas_call(
        _edge_body,
        grid=(nb,),
        in_specs=[
            pl.BlockSpec((bc, 128), lambda b: (b, 0)),
            pl.BlockSpec((16, be), lambda b: (0, b)),
            pl.BlockSpec((16, be), lambda b: (0, b)),
            small((32, 16)),
            small((16, 16)),
            small((16, 16)),
            small((64, 1)),
        ],
        out_specs=pl.BlockSpec((16, be), lambda b: (0, b)),
        out_shape=jax.ShapeDtypeStruct((16, e_total), jnp.float32),
    )(s128, et, rt, w12, w3, wr, bias_col)


def kernel(atom_fea, edge_ij, nbr_atoms, bonds_r, W1, b1, W2, b2, Wr, br, W3, b3):
    f = atom_fea.shape[1]
    e_total = edge_ij.shape[0]
    eq = e_total // 4
    wa = jnp.concatenate([W1[:, :f].T, W2[:, :f].T], axis=1)
    wb = jnp.concatenate([W1[:, f:2 * f].T, W2[:, f:2 * f].T], axis=1)
    w12 = jnp.concatenate([W1[:, 2 * f:], W2[:, 2 * f:]], axis=0)
    bias_col = jnp.concatenate([b1, b2, b3, br])[:, None]
    pa, pb = _node_proj(atom_fea, wa, wb)
    s128 = _gather_sum_sc(pa, pb, nbr_atoms.T)
    ot = _edge_mlp_t(s128, edge_ij.T, bonds_r.T, w12, W3, Wr, bias_col)
    return ot.T
